# baseline (speedup 1.0000x reference)
import functools

import jax
import jax.numpy as jnp
from jax import lax
from jax.experimental import pallas as pl
from jax.experimental.pallas import tpu as pltpu
from jax.experimental.pallas import tpu_sc as plsc

B = 128
N = 32768
NC = 2
NS = 16
NW = NC * NS

B_SC = 64
B_TC = B - B_SC

CSTR = 8
W = N // CSTR
W2 = W // 2
RGRP = NW // CSTR
RPW = B_SC // RGRP
RB = 4
NB = RPW // RB
NBP = NB // 2
L = 16

RT = 32
CT = 4096


def _body(x_hbm, w_hbm, b_hbm, out_hbm,
          xb0, xb1, ob0, ob1, wt, bt, wfi, bfi, si0, si1, so0, so1):
    wid = lax.axis_index("s") * NC + lax.axis_index("c")
    cs = wid % CSTR
    rg = wid // CSTR
    c0 = pl.multiple_of(cs * W, 8)
    h0 = pl.multiple_of(cs * W2, 8)
    r0 = rg * RPW

    nanv = jnp.full((L,), jnp.nan, dtype=jnp.float32)
    zerov = jnp.zeros((L,), dtype=jnp.float32)
    iota = lax.iota(jnp.int32, L)
    iota2 = iota * 2
    perm = iota ^ 1

    def in_copy(blk, xb, sem):
        row = r0 + blk * RB
        return pltpu.make_async_copy(
            x_hbm.at[pl.ds(row, RB), pl.ds(c0, W)], xb, sem)

    def out_copy(blk, ob, sem):
        row = r0 + blk * RB
        return pltpu.make_async_copy(
            ob, out_hbm.at[pl.ds(row, RB), 0, pl.ds(c0, W)], sem)

    def compute(xb, ob):
        @plsc.parallel_loop(0, W, step=L, unroll=4)
        def _vec(o):
            wv = wfi[pl.ds(o, L)]
            bv = bfi[pl.ds(o, L)]
            io = o + perm
            for r in range(RB):
                rv = jnp.full((L,), r, dtype=jnp.int32)
                xv = xb[r, pl.ds(o, L)]
                plsc.store_scatter(ob, [rv, io], xv * wv + bv)

    in_copy(0, xb0, si0).start()
    in_copy(1, xb1, si1).start()

    pltpu.sync_copy(w_hbm.at[pl.ds(h0, W2)], wt)
    pltpu.sync_copy(b_hbm.at[pl.ds(h0, W2)], bt)

    @plsc.parallel_loop(0, W, step=L, unroll=4)
    def _fill(o):
        wfi[pl.ds(o, L)] = zerov
        bfi[pl.ds(o, L)] = nanv

    @plsc.parallel_loop(0, W2, step=L, unroll=4)
    def _inter(j):
        ie = j * 2 + iota2
        plsc.store_scatter(wfi, [ie], wt[pl.ds(j, L)])
        plsc.store_scatter(bfi, [ie], bt[pl.ds(j, L)])

    @pl.loop(0, NBP)
    def _pair(g):
        blk0 = g * 2
        blk1 = blk0 + 1

        in_copy(blk0, xb0, si0).wait()

        @pl.when(g > 0)
        def _():
            out_copy(blk0 - 2, ob0, so0).wait()

        compute(xb0, ob0)
        out_copy(blk0, ob0, so0).start()

        @pl.when(g < NBP - 1)
        def _():
            in_copy(blk0 + 2, xb0, si0).start()

        in_copy(blk1, xb1, si1).wait()

        @pl.when(g > 0)
        def _():
            out_copy(blk1 - 2, ob1, so1).wait()

        compute(xb1, ob1)
        out_copy(blk1, ob1, so1).start()

        @pl.when(g < NBP - 1)
        def _():
            in_copy(blk1 + 2, xb1, si1).start()

    out_copy(NB - 2, ob0, so0).wait()
    out_copy(NB - 1, ob1, so1).wait()


def _tc_body(x_ref, wf_ref, bf_ref, prev_ref, o_ref):
    del prev_ref
    xr = pltpu.roll(x_ref[...], 1, 2)
    o_ref[...] = xr * wf_ref[...] + bf_ref[...]


def kernel(x, w, b):
    mesh = plsc.VectorSubcoreMesh(core_axis_name="c", subcore_axis_name="s")
    run = functools.partial(
        pl.kernel,
        out_type=jax.ShapeDtypeStruct((B, 1, N), jnp.float32),
        mesh=mesh,
        scratch_types=[
            pltpu.VMEM((RB, W), jnp.float32),
            pltpu.VMEM((RB, W), jnp.float32),
            pltpu.VMEM((RB, W), jnp.float32),
            pltpu.VMEM((RB, W), jnp.float32),
            pltpu.VMEM((W2,), jnp.float32),
            pltpu.VMEM((W2,), jnp.float32),
            pltpu.VMEM((W,), jnp.float32),
            pltpu.VMEM((W,), jnp.float32),
            pltpu.SemaphoreType.DMA,
            pltpu.SemaphoreType.DMA,
            pltpu.SemaphoreType.DMA,
            pltpu.SemaphoreType.DMA,
        ],
        compiler_params=pltpu.CompilerParams(needs_layout_passes=False),
    )(_body)
    sc_out = run(x, w, b)

    wf2 = jnp.stack([jnp.zeros_like(w), w], axis=1).reshape(1, N // 128, 128)
    bf2 = jnp.stack(
        [jnp.full_like(b, jnp.nan), b], axis=1).reshape(1, N // 128, 128)
    gr0 = B_SC // RT
    G = CT // 128
    out = pl.pallas_call(
        _tc_body,
        grid=(B_TC // RT, N // CT),
        in_specs=[
            pl.BlockSpec((RT, G, 128), lambda i, j: (gr0 + i, j, 0)),
            pl.BlockSpec((1, G, 128), lambda i, j: (0, j, 0)),
            pl.BlockSpec((1, G, 128), lambda i, j: (0, j, 0)),
            pl.BlockSpec(memory_space=pl.ANY),
        ],
        out_specs=pl.BlockSpec((RT, G, 128), lambda i, j: (gr0 + i, j, 0)),
        out_shape=jax.ShapeDtypeStruct((B, N // 128, 128), jnp.float32),
        input_output_aliases={3: 0},
    )(x.reshape(B, N // 128, 128), wf2, bf2,
      sc_out.reshape(B, N // 128, 128))
    return out.reshape(B, 1, N)


# speedup vs baseline: 2.3395x; 2.3395x over previous
import functools

import jax
import jax.numpy as jnp
from jax import lax
from jax.experimental import pallas as pl
from jax.experimental.pallas import tpu as pltpu
from jax.experimental.pallas import tpu_sc as plsc

B = 128
N = 32768
NC = 2
NS = 16
NW = NC * NS

CSTR = 8
W = N // CSTR
W2 = W // 2
RGRP = NW // CSTR
RPW = B // RGRP
RB = 4
NB = RPW // RB
NBP = NB // 2
L = 16


def _body(x_hbm, w_hbm, b_hbm, out_hbm,
          xb0, xb1, ob0, ob1, wt, bt, wfi, bfi, si0, si1, so0, so1):
    wid = lax.axis_index("s") * NC + lax.axis_index("c")
    cs = wid % CSTR
    rg = wid // CSTR
    c0 = pl.multiple_of(cs * W, 8)
    h0 = pl.multiple_of(cs * W2, 8)
    r0 = rg * RPW

    nanv = jnp.full((L,), jnp.nan, dtype=jnp.float32)
    zerov = jnp.zeros((L,), dtype=jnp.float32)
    iota = lax.iota(jnp.int32, L)
    iota2 = iota * 2
    perm = iota ^ 1

    def in_copy(blk, xb, sem):
        row = r0 + blk * RB
        return pltpu.make_async_copy(
            x_hbm.at[pl.ds(row, RB), pl.ds(c0, W)], xb, sem)

    def out_copy(blk, ob, sem):
        row = r0 + blk * RB
        return pltpu.make_async_copy(
            ob, out_hbm.at[pl.ds(row, RB), 0, pl.ds(c0, W)], sem)

    def compute(xb, ob):
        @plsc.parallel_loop(0, W, step=L, unroll=4)
        def _vec(o):
            wv = wfi[pl.ds(o, L)]
            bv = bfi[pl.ds(o, L)]
            io = o + perm
            for r in range(RB):
                rv = jnp.full((L,), r, dtype=jnp.int32)
                xv = xb[r, pl.ds(o, L)]
                plsc.store_scatter(ob, [rv, io], xv * wv + bv)

    in_copy(0, xb0, si0).start()
    in_copy(1, xb1, si1).start()

    pltpu.sync_copy(w_hbm.at[pl.ds(h0, W2)], wt)
    pltpu.sync_copy(b_hbm.at[pl.ds(h0, W2)], bt)

    @plsc.parallel_loop(0, W, step=L, unroll=4)
    def _fill(o):
        wfi[pl.ds(o, L)] = zerov
        bfi[pl.ds(o, L)] = nanv

    @plsc.parallel_loop(0, W2, step=L, unroll=4)
    def _inter(j):
        ie = j * 2 + iota2
        plsc.store_scatter(wfi, [ie], wt[pl.ds(j, L)])
        plsc.store_scatter(bfi, [ie], bt[pl.ds(j, L)])

    @pl.loop(0, NBP)
    def _pair(g):
        blk0 = g * 2
        blk1 = blk0 + 1

        in_copy(blk0, xb0, si0).wait()

        @pl.when(g > 0)
        def _():
            out_copy(blk0 - 2, ob0, so0).wait()

        compute(xb0, ob0)
        out_copy(blk0, ob0, so0).start()

        @pl.when(g < NBP - 1)
        def _():
            in_copy(blk0 + 2, xb0, si0).start()

        in_copy(blk1, xb1, si1).wait()

        @pl.when(g > 0)
        def _():
            out_copy(blk1 - 2, ob1, so1).wait()

        compute(xb1, ob1)
        out_copy(blk1, ob1, so1).start()

        @pl.when(g < NBP - 1)
        def _():
            in_copy(blk1 + 2, xb1, si1).start()

    out_copy(NB - 2, ob0, so0).wait()
    out_copy(NB - 1, ob1, so1).wait()


def kernel(x, w, b):
    mesh = plsc.VectorSubcoreMesh(core_axis_name="c", subcore_axis_name="s")
    run = functools.partial(
        pl.kernel,
        out_type=jax.ShapeDtypeStruct((B, 1, N), jnp.float32),
        mesh=mesh,
        scratch_types=[
            pltpu.VMEM((RB, W), jnp.float32),
            pltpu.VMEM((RB, W), jnp.float32),
            pltpu.VMEM((RB, W), jnp.float32),
            pltpu.VMEM((RB, W), jnp.float32),
            pltpu.VMEM((W2,), jnp.float32),
            pltpu.VMEM((W2,), jnp.float32),
            pltpu.VMEM((W,), jnp.float32),
            pltpu.VMEM((W,), jnp.float32),
            pltpu.SemaphoreType.DMA,
            pltpu.SemaphoreType.DMA,
            pltpu.SemaphoreType.DMA,
            pltpu.SemaphoreType.DMA,
        ],
        compiler_params=pltpu.CompilerParams(needs_layout_passes=False),
    )(_body)
    return run(x, w, b)
